# baseline (speedup 1.0000x reference)
import functools
import jax
import jax.numpy as jnp
from jax import lax
from jax.experimental import pallas as pl
from jax.experimental.pallas import tpu as pltpu
from jax.experimental.pallas import tpu_sc as plsc

N = 10000
NP = 10240
D = 128
E = 320000
NC, NS, L = 2, 16, 16
EPT = 20480
EPAD = EPT * NS
ROWS = EPT // 128
SLICE = NP // NS
PADIDX = NP - 1
f32 = jnp.float32


RB = 400
NB = N // RB


def _tc1_body(xl_ref, xr_ref, wl_ref, wr_ref, asvl_ref, asvr_ref,
              advl_ref, advr_ref,
              hl_ref, hr_ref, asrcl_ref, asrcr_ref, adstl_ref, adstr_ref,
              gvl_ref, gvr_ref, gm_ref):
    g = pl.program_id(0)
    i = pl.program_id(1)

    def do(x_ref, w_ref, asv_ref, adv_ref, h_ref, asrc_ref, adst_ref,
           gv_ref):
        h = jnp.dot(x_ref[...], w_ref[...], preferred_element_type=f32)
        h_ref[...] = h
        asrc = jnp.sum(h * asv_ref[...], axis=1, keepdims=True)
        adst = jnp.sum(h * adv_ref[...], axis=1, keepdims=True)
        asrc_ref[...] = asrc
        adst_ref[...] = adst
        m = jnp.max(asrc)

        @pl.when(i == 0)
        def _():
            gm_ref[0] = m

        @pl.when(i > 0)
        def _():
            gm_ref[0] = jnp.maximum(gm_ref[0], m)

        @pl.when(i == NB - 1)
        def _():
            gv_ref[...] = jnp.full((1, 16), gm_ref[0], f32)

    @pl.when(g == 0)
    def _():
        do(xl_ref, wl_ref, asvl_ref, advl_ref, hl_ref, asrcl_ref,
           adstl_ref, gvl_ref)

    @pl.when(g == 1)
    def _():
        do(xr_ref, wr_ref, asvr_ref, advr_ref, hr_ref, asrcr_ref,
           adstr_ref, gvr_ref)


def _tc1(xl, xr, Wl, Wr, asl, asr, adl, adr):
    const = lambda g, i: (0, 0)
    lmap = lambda g, i: (jnp.where(g == 0, i, NB - 1), 0)
    rmap = lambda g, i: (jnp.where(g == 0, 0, i), 0)
    return pl.pallas_call(
        _tc1_body,
        grid=(2, NB),
        in_specs=[
            pl.BlockSpec((RB, D), lmap),
            pl.BlockSpec((RB, D), rmap),
            pl.BlockSpec((D, D), const),
            pl.BlockSpec((D, D), const),
            pl.BlockSpec((1, D), const),
            pl.BlockSpec((1, D), const),
            pl.BlockSpec((1, D), const),
            pl.BlockSpec((1, D), const),
        ],
        out_specs=[
            pl.BlockSpec((RB, D), lmap),
            pl.BlockSpec((RB, D), rmap),
            pl.BlockSpec((RB, 1), lmap),
            pl.BlockSpec((RB, 1), rmap),
            pl.BlockSpec((RB, 1), lmap),
            pl.BlockSpec((RB, 1), rmap),
            pl.BlockSpec((1, 16), const),
            pl.BlockSpec((1, 16), const),
        ],
        out_shape=[
            jax.ShapeDtypeStruct((N, D), f32),
            jax.ShapeDtypeStruct((N, D), f32),
            jax.ShapeDtypeStruct((N, 1), f32),
            jax.ShapeDtypeStruct((N, 1), f32),
            jax.ShapeDtypeStruct((N, 1), f32),
            jax.ShapeDtypeStruct((N, 1), f32),
            jax.ShapeDtypeStruct((1, 16), f32),
            jax.ShapeDtypeStruct((1, 16), f32),
        ],
        scratch_shapes=[pltpu.SMEM((1,), f32)],
    )(xl, xr, Wl, Wr, asl.reshape(1, D), asr.reshape(1, D),
      adl.reshape(1, D), adr.reshape(1, D))


def _lrelu(v):
    return jnp.maximum(v, 0.2 * v)


CHUNK = 8


def _sc_graph(sid, asrc_h, adst_h, gmax_h, src_h, dst_h, w_h,
              asrc_v, adst_v, dinv_v, gmax_v, src_v, dst_v, tbuf_v,
              sl_a, sl_b, acc_sh, sem):
    descs = [
        pltpu.async_copy(asrc_h, asrc_v.at[pl.ds(0, N)], sem),
        pltpu.async_copy(adst_h, adst_v.at[pl.ds(0, N)], sem),
        pltpu.async_copy(gmax_h, gmax_v, sem),
        pltpu.async_copy(src_h.at[sid], src_v, sem),
        pltpu.async_copy(dst_h.at[sid], dst_v, sem),
    ]
    for dsc in descs:
        dsc.wait()

    z16 = jnp.zeros((L,), f32)

    def zloop(k, _):
        sl_a[pl.ds(k * L, L)] = z16
        return 0

    lax.fori_loop(0, SLICE // L, zloop, 0)
    pltpu.sync_copy(sl_a, acc_sh.at[pl.ds(sid * SLICE, SLICE)])
    plsc.subcore_barrier()

    gv = gmax_v[...]

    @plsc.parallel_loop(0, ROWS, 1, unroll=2)
    def p1(r):
        for c in range(128 // L):
            s16 = src_v[r, pl.ds(c * L, L)]
            d16 = dst_v[r, pl.ds(c * L, L)]
            a_s = plsc.load_gather(asrc_v, [s16])
            a_d = plsc.load_gather(adst_v, [d16])
            e = _lrelu(a_s + a_d)
            cc = _lrelu(gv + a_d)
            tbuf_v[r, pl.ds(c * L, L)] = jnp.exp(e - cc)

    def p1s(cnk, _):
        base = cnk * CHUNK
        ds_ = [pltpu.async_copy(tbuf_v.at[base + j],
                                acc_sh.at[dst_v.at[base + j]], sem, add=True)
               for j in range(CHUNK)]
        for dsc in ds_:
            dsc.wait()
        return 0

    lax.fori_loop(0, ROWS // CHUNK, p1s, 0)
    plsc.subcore_barrier()

    pltpu.sync_copy(acc_sh.at[pl.ds(sid * SLICE, SLICE)], sl_a)

    def dloop(k, _):
        a_s = asrc_v[pl.ds(sid * SLICE + k * L, L)]
        a_d = adst_v[pl.ds(sid * SLICE + k * L, L)]
        dinit = jnp.exp(_lrelu(a_s + a_d) - _lrelu(gv + a_d))
        den = sl_a[pl.ds(k * L, L)] + dinit
        dinv = 1.0 / (den + 1e-16)
        sl_a[pl.ds(k * L, L)] = dinv
        sl_b[pl.ds(k * L, L)] = dinit * dinv
        return 0

    lax.fori_loop(0, SLICE // L, dloop, 0)
    pltpu.sync_copy(sl_a, acc_sh.at[pl.ds(sid * SLICE, SLICE)])
    plsc.subcore_barrier()
    pltpu.sync_copy(acc_sh, dinv_v)
    plsc.subcore_barrier()

    lax.fori_loop(0, SLICE // L, zloop, 0)
    pltpu.sync_copy(sl_a, acc_sh.at[pl.ds(sid * SLICE, SLICE)])
    plsc.subcore_barrier()

    @plsc.parallel_loop(0, ROWS, 1, unroll=2)
    def p2(r):
        for c in range(128 // L):
            d16 = dst_v[r, pl.ds(c * L, L)]
            di = plsc.load_gather(dinv_v, [d16])
            t = tbuf_v[r, pl.ds(c * L, L)]
            tbuf_v[r, pl.ds(c * L, L)] = t * di

    def p2s(cnk, _):
        base = cnk * CHUNK
        ds_ = [pltpu.async_copy(tbuf_v.at[base + j],
                                acc_sh.at[src_v.at[base + j]], sem, add=True)
               for j in range(CHUNK)]
        for dsc in ds_:
            dsc.wait()
        return 0

    lax.fori_loop(0, ROWS // CHUNK, p2s, 0)
    plsc.subcore_barrier()

    pltpu.sync_copy(acc_sh.at[pl.ds(sid * SLICE, SLICE)], sl_a)

    def wloop(k, _):
        node0 = sid * SLICE + k * L
        keep = (lax.iota(jnp.int32, L) + node0) < N
        w16 = sl_a[pl.ds(k * L, L)] + sl_b[pl.ds(k * L, L)]
        sl_a[pl.ds(k * L, L)] = jnp.where(keep, w16, 0.0)
        return 0

    lax.fori_loop(0, SLICE // L, wloop, 0)
    pltpu.sync_copy(sl_a, w_h.at[pl.ds(sid * SLICE, SLICE)])


def _make_sc_kernel():
    mesh = plsc.VectorSubcoreMesh(core_axis_name="c", subcore_axis_name="s")

    @functools.partial(
        pl.kernel,
        out_type=[jax.ShapeDtypeStruct((NP,), f32),
                  jax.ShapeDtypeStruct((NP,), f32)],
        mesh=mesh,
        compiler_params=pltpu.CompilerParams(needs_layout_passes=False),
        scratch_types=[
            pltpu.VMEM((NP,), f32),
            pltpu.VMEM((NP,), f32),
            pltpu.VMEM((NP,), f32),
            pltpu.VMEM((L,), f32),
            pltpu.VMEM((ROWS, 128), jnp.int32),
            pltpu.VMEM((ROWS, 128), jnp.int32),
            pltpu.VMEM((ROWS, 128), f32),
            pltpu.VMEM((SLICE,), f32),
            pltpu.VMEM((SLICE,), f32),
            pltpu.VMEM_SHARED((NP,), f32),
            pltpu.SemaphoreType.DMA,
        ],
    )
    def sc_kernel(asrc_l, adst_l, gmax_l, src_l, dst_l,
                  asrc_r, adst_r, gmax_r, src_r, dst_r,
                  w_l, w_r,
                  asrc_v, adst_v, dinv_v, gmax_v, src_v, dst_v, tbuf_v,
                  sl_a, sl_b, acc_sh, sem):
        cid = lax.axis_index("c")
        sid = lax.axis_index("s")

        @pl.when(cid == 0)
        def _():
            _sc_graph(sid, asrc_l, adst_l, gmax_l, src_l, dst_l, w_l,
                      asrc_v, adst_v, dinv_v, gmax_v, src_v, dst_v, tbuf_v,
                      sl_a, sl_b, acc_sh, sem)

        @pl.when(cid == 1)
        def _():
            _sc_graph(sid, asrc_r, adst_r, gmax_r, src_r, dst_r, w_r,
                      asrc_v, adst_v, dinv_v, gmax_v, src_v, dst_v, tbuf_v,
                      sl_a, sl_b, acc_sh, sem)

    return sc_kernel


_sc_kernel = _make_sc_kernel()


CH = 1024
NCHUNK = (D * D) // CH


def _tc2_body(wl_ref, hl_ref, wr_ref, hr_ref, bl_ref, br_ref,
              w1_ref, b1_ref, w2_ref, b2_ref, w3_ref, b3_ref,
              out_ref, feat_ref, acc_ref):
    j = pl.program_id(0)

    @pl.when(j == 0)
    def _():
        wl = lax.slice(wl_ref[...], (0, 0), (1, N))
        wr = lax.slice(wr_ref[...], (0, 0), (1, N))
        al = jnp.dot(wl, hl_ref[...], preferred_element_type=f32)
        ar = jnp.dot(wr, hr_ref[...], preferred_element_type=f32)
        feat_ref[:, 0:D] = al + N * bl_ref[...]
        feat_ref[:, D:2 * D] = ar + N * br_ref[...]
        acc_ref[...] = jnp.zeros_like(acc_ref)

    h1 = jnp.dot(feat_ref[...], w1_ref[...], preferred_element_type=f32)
    h1 = h1 + b1_ref[...]
    acc_ref[...] += jnp.dot(h1, w2_ref[...], preferred_element_type=f32)

    @pl.when(j == NCHUNK - 1)
    def _():
        h2 = acc_ref[...] + b2_ref[...]
        out_ref[...] = jnp.dot(h2, w3_ref[...], preferred_element_type=f32) \
            + b3_ref[...]


def _tc2(wl, hl, wr, hr, bl, br, W1, b1, W2, b2, W3, b3):
    const = lambda *_: (0, 0)
    return pl.pallas_call(
        _tc2_body,
        grid=(NCHUNK,),
        in_specs=[
            pl.BlockSpec((1, NP), const),
            pl.BlockSpec((N, D), const),
            pl.BlockSpec((1, NP), const),
            pl.BlockSpec((N, D), const),
            pl.BlockSpec((1, D), const),
            pl.BlockSpec((1, D), const),
            pl.BlockSpec((2 * D, CH), lambda j: (0, j)),
            pl.BlockSpec((1, CH), lambda j: (0, j)),
            pl.BlockSpec((CH, D), lambda j: (j, 0)),
            pl.BlockSpec((1, D), const),
            pl.BlockSpec((D, 1), const),
            pl.BlockSpec((1, 1), const),
        ],
        out_specs=pl.BlockSpec((1, 1), const),
        out_shape=jax.ShapeDtypeStruct((1, 1), f32),
        scratch_shapes=[
            pltpu.VMEM((1, 2 * D), f32),
            pltpu.VMEM((1, D), f32),
        ],
    )(wl, hl, wr, hr, bl, br, W1, b1, W2, b2, W3, b3)


def _prep_edges(ei):
    ei = ei.astype(jnp.int32)
    pad = jnp.full((2, EPAD - E), PADIDX, jnp.int32)
    eip = jnp.concatenate([ei, pad], axis=1)
    return (eip[0].reshape(NS, ROWS, 128),
            eip[1].reshape(NS, ROWS, 128))


def kernel(l_x, l_edge_index, r_x, r_edge_index,
           Wl, att_src_l, att_dst_l, bl,
           Wr, att_src_r, att_dst_r, br,
           W1, b1, W2, b2, W3, b3):
    hl, hr, asl, asr, adl, adr, gvl, gvr = _tc1(
        l_x, r_x, Wl, Wr, att_src_l, att_src_r, att_dst_l, att_dst_r)

    s_l, d_l = _prep_edges(l_edge_index)
    s_r, d_r = _prep_edges(r_edge_index)

    wl_, wr_ = _sc_kernel(asl.reshape(N), adl.reshape(N), gvl.reshape(L),
                          s_l, d_l,
                          asr.reshape(N), adr.reshape(N), gvr.reshape(L),
                          s_r, d_r)

    out = _tc2(wl_.reshape(1, NP), hl, wr_.reshape(1, NP), hr,
               bl.reshape(1, D), br.reshape(1, D),
               W1, b1.reshape(1, D * D), W2, b2.reshape(1, D),
               W3, b3.reshape(1, 1))
    return out.reshape(1)
